# Initial kernel scaffold; baseline (speedup 1.0000x reference)
import jax
import jax.numpy as jnp
from jax.experimental import pallas as pl


def kernel(x, params, edge_index, batch):
    raise NotImplementedError("write your pallas kernel here")


# speedup vs baseline: 10.2490x; 10.2490x over previous
import functools

import jax
import jax.numpy as jnp
from jax import lax
from jax.experimental import pallas as pl
from jax.experimental.pallas import tpu as pltpu
from jax.experimental.pallas import tpu_sc as plsc

N = 10000
E = 320000
D_IN = 128
D_H = 32
D_OUT = 128
G = 128

NC = 2
NS = 16
NW = NC * NS

B = 128
K = -(-E // (NW * B))
EPW = K * B
EPAD = NW * EPW

ZR = 672
N_ACC = NS * ZR

_f32 = jnp.float32
_HI = lax.Precision.HIGHEST

_mesh = plsc.VectorSubcoreMesh(core_axis_name="c", subcore_axis_name="s")


def _make_sc_agg(width):
    @functools.partial(
        pl.kernel,
        out_type=jax.ShapeDtypeStruct((NC, N_ACC, width), _f32),
        mesh=_mesh,
        compiler_params=pltpu.CompilerParams(use_tc_tiling_on_sc=False),
        scratch_types=[
            pltpu.VMEM((K, B), jnp.int32),
            pltpu.VMEM((K, B), jnp.int32),
            pltpu.VMEM((B, width), _f32),
            pltpu.VMEM_SHARED((N_ACC, width), _f32),
        ],
    )
    def sc_agg(q_hbm, src_hbm, dst_hbm, zer_hbm, out_hbm, src_v, dst_v,
               rows_v, acc_sh):
        c = lax.axis_index("c")
        s = lax.axis_index("s")
        wid = s * NC + c

        pltpu.sync_copy(zer_hbm, acc_sh.at[pl.ds(s * ZR, ZR)])
        pltpu.sync_copy(src_hbm.at[wid], src_v)
        pltpu.sync_copy(dst_hbm.at[wid], dst_v)
        plsc.subcore_barrier()

        def body(j, carry):
            pltpu.sync_copy(q_hbm.at[src_v.at[j]], rows_v)
            pltpu.sync_copy(rows_v, acc_sh.at[dst_v.at[j]], add=True)
            return carry

        lax.fori_loop(0, K, body, 0)
        plsc.subcore_barrier()

        pltpu.sync_copy(acc_sh.at[pl.ds(s * ZR, ZR)],
                        out_hbm.at[c, pl.ds(s * ZR, ZR)])

    return sc_agg


_sc_agg128 = _make_sc_agg(D_IN)
_sc_agg32 = _make_sc_agg(D_H)


def _mid_body(h_ref, acc_ref, w1_ref, b1_ref, w2_ref, b2_ref, g_ref, be_ref,
              o_ref):
    _bf = jnp.bfloat16
    z = h_ref[...] + acc_ref[0, :N] + acc_ref[1, :N]
    u = jnp.dot(z.astype(_bf), w1_ref[...].astype(_bf),
                preferred_element_type=_f32) + b1_ref[...]
    t = jnp.maximum(u, 0.0)
    v = jnp.dot(t.astype(_bf), w2_ref[...].astype(_bf),
                preferred_element_type=_f32) + b2_ref[...]
    h = jnp.maximum(v, 0.0)
    mean = jnp.mean(h, axis=0, keepdims=True)
    d = h - mean
    var = jnp.mean(d * d, axis=0, keepdims=True)
    o_ref[...] = d / jnp.sqrt(var + 1e-5) * g_ref[...] + be_ref[...]


def _make_mid(width):
    return pl.pallas_call(
        _mid_body,
        out_shape=jax.ShapeDtypeStruct((N, D_H), _f32),
    )


_mid128 = _make_mid(D_IN)
_mid32 = _make_mid(D_H)


def _final_body(h_ref, bt_ref, fw_ref, fb_ref, o_ref):
    hn = h_ref[...]
    bm = jnp.broadcast_to(bt_ref[...], (G, N))
    gi = lax.broadcasted_iota(jnp.int32, (G, N), 0).astype(_f32)
    onehot = jnp.where(bm == gi, 1.0, 0.0)
    pooled = jnp.dot(onehot, hn, precision=_HI, preferred_element_type=_f32)
    o_ref[...] = jnp.maximum(
        jnp.dot(pooled, fw_ref[...], preferred_element_type=_f32)
        + fb_ref[...], 0.0)


_final = pl.pallas_call(
    _final_body,
    out_shape=jax.ShapeDtypeStruct((G, D_OUT), _f32),
)


def kernel(x, params, edge_index, batch):
    src = edge_index[0]
    dst = edge_index[1]
    pad = EPAD - E
    pi = jnp.arange(pad, dtype=jnp.int32)
    src_p = jnp.concatenate([src, pi % N]).reshape(NW, K, B)
    dst_p = jnp.concatenate([dst, N + pi % (N_ACC - N)]).reshape(NW, K, B)
    zer128 = jnp.zeros((ZR, D_IN), _f32)
    zer32 = jnp.zeros((ZR, D_H), _f32)
    bt = batch.astype(_f32).reshape(1, N)

    h = x
    for i in range(1, 6):
        cp = params['conv%d' % i]
        bp = params['bn%d' % i]
        if i == 1:
            acc = _sc_agg128(h, src_p, dst_p, zer128)
            mid = _mid128
        else:
            acc = _sc_agg32(h, src_p, dst_p, zer32)
            mid = _mid32
        h = mid(h, acc,
                cp['w1'], cp['b1'].reshape(1, D_H),
                cp['w2'], cp['b2'].reshape(1, D_H),
                bp['gamma'].reshape(1, D_H), bp['beta'].reshape(1, D_H))
    return _final(h, bt, params['fc1']['w'],
                  params['fc1']['b'].reshape(1, D_OUT))
